# baseline (speedup 1.0000x reference)
import functools

import jax
import jax.numpy as jnp
from jax.experimental import pallas as pl

_B, _E, _C = 16384, 4096, 6
_BLK = 1024


def _proposal_kernel(x_ref, wt_ref, b_ref, g_ref, out_ref):
    logits = jax.lax.dot_general(
        x_ref[...], wt_ref[...],
        dimension_numbers=(((1,), (0,)), ((), ())),
        preferred_element_type=jnp.float32,
        precision=jax.lax.Precision.DEFAULT,
    ) + b_ref[...]
    m = jnp.max(logits, axis=-1, keepdims=True)
    e = jnp.exp(logits - m)
    p = e / jnp.sum(e, axis=-1, keepdims=True)
    v = jnp.log(p + 1e-12) + g_ref[...]
    out_ref[...] = jnp.argmax(v, axis=-1).astype(jnp.int32)


@functools.partial(jax.jit, static_argnames=())
def kernel(x, W, b):
    wt = W.T
    gumbel = jax.random.gumbel(jax.random.key(42), (_B, _C), jnp.float32)
    grid = (_B // _BLK,)
    return pl.pallas_call(
        _proposal_kernel,
        grid=grid,
        in_specs=[
            pl.BlockSpec((_BLK, _E), lambda i: (i, 0)),
            pl.BlockSpec((_E, _C), lambda i: (0, 0)),
            pl.BlockSpec((1, _C), lambda i: (0, 0)),
            pl.BlockSpec((_BLK, _C), lambda i: (i, 0)),
        ],
        out_specs=pl.BlockSpec((_BLK,), lambda i: (i,)),
        out_shape=jax.ShapeDtypeStruct((_B,), jnp.int32),
    )(x, wt, b.reshape(1, _C), gumbel)
